# Initial kernel scaffold; baseline (speedup 1.0000x reference)
import functools

import jax
import jax.numpy as jnp
from jax import lax
from jax.experimental import pallas as pl
from jax.experimental.pallas import tpu as pltpu

_MASK_THR = 0.45
_SIGMA = 2.0


def _prep_body(n, T, seg_ref, out_ref):
    i = pl.program_id(0)
    x = seg_ref[...]
    row = i * T + lax.broadcasted_iota(jnp.int32, x.shape, 0)
    m = (x > _MASK_THR) & (row < n)
    out_ref[...] = m.astype(jnp.bfloat16)


def _tile_d(T, a, b, ma, mb, sa, sb, la, lb):
    inter = lax.dot_general(ma, mb, (((1,), (1,)), ((), ())),
                            preferred_element_type=jnp.float32)
    iou = inter / (sa[:, None] + sb[None, :] - inter)
    ga = a * T + lax.broadcasted_iota(jnp.int32, inter.shape, 0)
    gb = b * T + lax.broadcasted_iota(jnp.int32, inter.shape, 1)
    valid = (la[:, None] == lb[None, :]) & (ga < gb)
    return jnp.where(valid, iou, 0.0)


def _pass1_body(T, ma_ref, mb_ref, sa_ref, sb_ref, la_ref, lb_ref, c_ref):
    b = pl.program_id(0)
    a = pl.program_id(1)

    @pl.when(a == 0)
    def _():
        c_ref[...] = jnp.zeros_like(c_ref)

    @pl.when(a <= b)
    def _():
        d = _tile_d(T, a, b, ma_ref[...], mb_ref[...], sa_ref[...],
                    sb_ref[...], la_ref[...], lb_ref[...])
        c_ref[...] = jnp.maximum(c_ref[...], jnp.max(d, axis=0))


def _pass2_body(T, nb, ma_ref, mb_ref, sa_ref, sb_ref, la_ref, lb_ref,
                ca_ref, m_ref):
    b = pl.program_id(0)
    a = pl.program_id(1)

    @pl.when(a == 0)
    def _():
        m_ref[...] = jnp.zeros_like(m_ref)

    @pl.when(a <= b)
    def _():
        d = _tile_d(T, a, b, ma_ref[...], mb_ref[...], sa_ref[...],
                    sb_ref[...], la_ref[...], lb_ref[...])
        ca = ca_ref[...]
        term = d * d - (ca * ca)[:, None]
        m_ref[...] = jnp.maximum(m_ref[...], jnp.max(term, axis=0))

    @pl.when(a == nb - 1)
    def _():
        m_ref[...] = jnp.exp(-_SIGMA * m_ref[...])


def _prep(seg_rows_s, P, T, hw):
    n = seg_rows_s.shape[0]
    nb = P // T
    return pl.pallas_call(
        functools.partial(_prep_body, n, T),
        grid=(nb,),
        in_specs=[pl.BlockSpec((T, hw), lambda i: (i, 0))],
        out_specs=pl.BlockSpec((T, hw), lambda i: (i, 0)),
        out_shape=jax.ShapeDtypeStruct((P, hw), jnp.bfloat16),
    )(seg_rows_s)


def _nms_core(masks, sums, labels, T):
    P, hw = masks.shape
    nb = P // T
    grid = (nb, nb)
    mspec_a = pl.BlockSpec((T, hw), lambda b, a: (jnp.minimum(a, b), 0))
    mspec_b = pl.BlockSpec((T, hw), lambda b, a: (b, 0))
    vspec_a = pl.BlockSpec((T,), lambda b, a: (jnp.minimum(a, b),))
    vspec_b = pl.BlockSpec((T,), lambda b, a: (b,))
    params = pltpu.CompilerParams(dimension_semantics=("arbitrary", "arbitrary"))
    c = pl.pallas_call(
        functools.partial(_pass1_body, T),
        grid=grid,
        in_specs=[mspec_a, mspec_b, vspec_a, vspec_b, vspec_a, vspec_b],
        out_specs=pl.BlockSpec((T,), lambda b, a: (b,)),
        out_shape=jax.ShapeDtypeStruct((P,), jnp.float32),
        compiler_params=params,
    )(masks, masks, sums, sums, labels, labels)
    coeff = pl.pallas_call(
        functools.partial(_pass2_body, T, nb),
        grid=grid,
        in_specs=[mspec_a, mspec_b, vspec_a, vspec_b, vspec_a, vspec_b,
                  vspec_a],
        out_specs=pl.BlockSpec((T,), lambda b, a: (b,)),
        out_shape=jax.ShapeDtypeStruct((P,), jnp.float32),
        compiler_params=params,
    )(masks, masks, sums, sums, labels, labels, c)
    return coeff


def kernel(seg_preds, cate_scores, cate_labels):
    n, h, w = seg_preds.shape
    hw = h * w
    seg_masks_b = seg_preds > _MASK_THR
    seg_masks_f = seg_masks_b.astype(jnp.float32)
    sum_masks = seg_masks_f.reshape(n, -1).sum(axis=1)
    seg_scores = (seg_preds * seg_masks_f).reshape(n, -1).sum(axis=1) / sum_masks
    cs = cate_scores * seg_scores
    sort_inds = jnp.argsort(-cs)

    T = 640
    P = ((n + T - 1) // T) * T
    pad = P - n

    seg_rows = seg_preds.reshape(n, hw)
    seg_rows_s = jnp.take(seg_rows, sort_inds, axis=0)
    masks = _prep(seg_rows_s, P, T, hw)
    sums_p = jnp.pad(jnp.take(sum_masks, sort_inds), (0, pad),
                     constant_values=1.0)
    labels_p = jnp.pad(jnp.take(cate_labels, sort_inds), (0, pad),
                       constant_values=-1)
    coeff = _nms_core(masks, sums_p, labels_p, T)
    scores_s = jnp.take(cs, sort_inds)
    return (seg_rows_s.reshape(n, h, w),
            scores_s * coeff[:n],
            jnp.take(cate_labels, sort_inds))


# speedup vs baseline: 2.1497x; 2.1497x over previous
import functools

import jax
import jax.numpy as jnp
from jax import lax
from jax.experimental import pallas as pl
from jax.experimental.pallas import tpu as pltpu

_MASK_THR = 0.45
_SIGMA = 2.0


def _prep_body(n, T, seg_ref, out_ref):
    i = pl.program_id(0)
    x = seg_ref[...]
    row = i * T + lax.broadcasted_iota(jnp.int32, x.shape, 0)
    m = (x > _MASK_THR) & (row < n)
    out_ref[...] = m.astype(jnp.bfloat16)


def _tile_d(T, a, b, ma, mb, sa, sb, la, lb):
    inter = lax.dot_general(ma, mb, (((1,), (1,)), ((), ())),
                            preferred_element_type=jnp.float32)
    iou = inter / (sa[:, None] + sb[None, :] - inter)
    ga = a * T + lax.broadcasted_iota(jnp.int32, inter.shape, 0)
    gb = b * T + lax.broadcasted_iota(jnp.int32, inter.shape, 1)
    valid = (la[:, None] == lb[None, :]) & (ga < gb)
    return jnp.where(valid, iou, 0.0)


def _pass1_body(T, ma_ref, mb_ref, sa_ref, sb_ref, la_ref, lb_ref, c_ref):
    b = pl.program_id(0)
    a = pl.program_id(1)

    @pl.when(a == 0)
    def _():
        c_ref[...] = jnp.zeros_like(c_ref)

    @pl.when(a <= b)
    def _():
        d = _tile_d(T, a, b, ma_ref[...], mb_ref[...], sa_ref[0, 0, :],
                    sb_ref[0, 0, :], la_ref[0, 0, :], lb_ref[0, 0, :])
        c_ref[0, 0, :] = jnp.maximum(c_ref[0, 0, :], jnp.max(d, axis=0))


def _pass2_body(T, nb, ma_ref, mb_ref, sa_ref, sb_ref, la_ref, lb_ref,
                ca_ref, m_ref):
    b = pl.program_id(0)
    a = pl.program_id(1)

    @pl.when(a == 0)
    def _():
        m_ref[...] = jnp.zeros_like(m_ref)

    @pl.when(a <= b)
    def _():
        d = _tile_d(T, a, b, ma_ref[...], mb_ref[...], sa_ref[0, 0, :],
                    sb_ref[0, 0, :], la_ref[0, 0, :], lb_ref[0, 0, :])
        ca = ca_ref[0, 0, :]
        term = d * d - (ca * ca)[:, None]
        m_ref[0, 0, :] = jnp.maximum(m_ref[0, 0, :], jnp.max(term, axis=0))

    @pl.when(a == nb - 1)
    def _():
        m_ref[...] = jnp.exp(-_SIGMA * m_ref[...])


def _prep(seg_rows_s, P, T, hw):
    n = seg_rows_s.shape[0]
    nb = P // T
    return pl.pallas_call(
        functools.partial(_prep_body, n, T),
        grid=(nb,),
        in_specs=[pl.BlockSpec((T, hw), lambda i: (i, 0))],
        out_specs=pl.BlockSpec((T, hw), lambda i: (i, 0)),
        out_shape=jax.ShapeDtypeStruct((P, hw), jnp.bfloat16),
    )(seg_rows_s)


def _nms_core(masks, sums, labels, T):
    P, hw = masks.shape
    nb = P // T
    grid = (nb, nb)
    sums3 = sums.reshape(nb, 1, T)
    labels3 = labels.reshape(nb, 1, T)
    mspec_a = pl.BlockSpec((T, hw), lambda b, a: (jnp.minimum(a, b), 0))
    mspec_b = pl.BlockSpec((T, hw), lambda b, a: (b, 0))
    vspec_a = pl.BlockSpec((1, 1, T), lambda b, a: (jnp.minimum(a, b), 0, 0))
    vspec_b = pl.BlockSpec((1, 1, T), lambda b, a: (b, 0, 0))
    params = pltpu.CompilerParams(dimension_semantics=("arbitrary", "arbitrary"))
    c = pl.pallas_call(
        functools.partial(_pass1_body, T),
        grid=grid,
        in_specs=[mspec_a, mspec_b, vspec_a, vspec_b, vspec_a, vspec_b],
        out_specs=pl.BlockSpec((1, 1, T), lambda b, a: (b, 0, 0)),
        out_shape=jax.ShapeDtypeStruct((nb, 1, T), jnp.float32),
        compiler_params=params,
    )(masks, masks, sums3, sums3, labels3, labels3)
    coeff = pl.pallas_call(
        functools.partial(_pass2_body, T, nb),
        grid=grid,
        in_specs=[mspec_a, mspec_b, vspec_a, vspec_b, vspec_a, vspec_b,
                  vspec_a],
        out_specs=pl.BlockSpec((1, 1, T), lambda b, a: (b, 0, 0)),
        out_shape=jax.ShapeDtypeStruct((nb, 1, T), jnp.float32),
        compiler_params=params,
    )(masks, masks, sums3, sums3, labels3, labels3, c)
    return coeff.reshape(P)


def kernel(seg_preds, cate_scores, cate_labels):
    n, h, w = seg_preds.shape
    hw = h * w
    seg_masks_b = seg_preds > _MASK_THR
    seg_masks_f = seg_masks_b.astype(jnp.float32)
    sum_masks = seg_masks_f.reshape(n, -1).sum(axis=1)
    seg_scores = (seg_preds * seg_masks_f).reshape(n, -1).sum(axis=1) / sum_masks
    cs = cate_scores * seg_scores
    sort_inds = jnp.argsort(-cs)

    T = 640
    P = ((n + T - 1) // T) * T
    pad = P - n

    seg_rows = seg_preds.reshape(n, hw)
    seg_rows_s = jnp.take(seg_rows, sort_inds, axis=0)
    masks = _prep(seg_rows_s, P, T, hw)
    sums_p = jnp.pad(jnp.take(sum_masks, sort_inds), (0, pad),
                     constant_values=1.0)
    labels_p = jnp.pad(jnp.take(cate_labels, sort_inds), (0, pad),
                       constant_values=-1)
    coeff = _nms_core(masks, sums_p, labels_p, T)
    scores_s = jnp.take(cs, sort_inds)
    return (seg_rows_s.reshape(n, h, w),
            scores_s * coeff[:n],
            jnp.take(cate_labels, sort_inds))
